# baseline (speedup 1.0000x reference)
import jax
import jax.numpy as jnp
from jax.experimental import pallas as pl


def _pack_body(x_ref, fast_ref, slow_ref):
    v = x_ref[...]
    fast_ref[...] = v
    slow_ref[...] = v


def kernel(frames):
    C, T, H, W = frames.shape
    S = T // 4
    HW = H * W
    L = 128
    R = HW // L
    x = frames.reshape(C, T, R, L)

    def slow_map(i):
        return (0, ((S - 1) * i + (T - 2)) // (T - 1), 0, 0)

    slow, fast = pl.pallas_call(
        _pack_body,
        grid=(T,),
        in_specs=[pl.BlockSpec((C, 1, R, L), lambda i: (0, i, 0, 0))],
        out_specs=[
            pl.BlockSpec((C, 1, R, L), slow_map),
            pl.BlockSpec((C, 1, R, L), lambda i: (0, i, 0, 0)),
        ],
        out_shape=[
            jax.ShapeDtypeStruct((C, S, R, L), frames.dtype),
            jax.ShapeDtypeStruct((C, T, R, L), frames.dtype),
        ],
    )(x)
    return (slow.reshape(C, S, H, W), fast.reshape(C, T, H, W))
